# baseline (speedup 1.0000x reference)
import functools

import jax
import jax.numpy as jnp
from jax import lax
from jax.experimental import pallas as pl
from jax.experimental.pallas import tpu as pltpu
from jax.experimental.pallas import tpu_sc as plsc

_BATCH = 16384
_HIST = 50
_DIM = 32
_N = _BATCH * _HIST
_NC = 2
_NS = 16
_NW = _NC * _NS
_PER_W = _N // _NW
_CHUNK = 1600
_NCHUNK = _PER_W // _CHUNK


@functools.partial(
    pl.kernel,
    out_type=jax.ShapeDtypeStruct((_N, _DIM), jnp.float32),
    mesh=plsc.VectorSubcoreMesh(core_axis_name="c", subcore_axis_name="s"),
    scratch_types=[
        pltpu.VMEM((_CHUNK,), jnp.int32),
        pltpu.VMEM((_CHUNK,), jnp.int32),
        pltpu.VMEM((_CHUNK, _DIM), jnp.float32),
        pltpu.VMEM((_CHUNK, _DIM), jnp.float32),
        pltpu.SemaphoreType.DMA,
        pltpu.SemaphoreType.DMA,
        pltpu.SemaphoreType.DMA,
        pltpu.SemaphoreType.DMA,
        pltpu.SemaphoreType.DMA,
        pltpu.SemaphoreType.DMA,
    ],
    compiler_params=pltpu.CompilerParams(use_tc_tiling_on_sc=False),
)
def _gather_kernel(idx_hbm, table_hbm, out_hbm, idx0, idx1, rows0, rows1,
                   i0, i1, g0, g1, s0, s1):
    wid = lax.axis_index("s") * _NC + lax.axis_index("c")
    base = wid * _PER_W
    idx_v = (idx0, idx1)
    rows = (rows0, rows1)
    isem = (i0, i1)
    gsem = (g0, g1)
    ssem = (s0, s1)

    def idx_load(c):
        return pltpu.async_copy(
            idx_hbm.at[pl.ds(base + c * _CHUNK, _CHUNK)],
            idx_v[c % 2], isem[c % 2])

    def gather(c):
        return pltpu.async_copy(
            table_hbm.at[idx_v[c % 2]], rows[c % 2], gsem[c % 2])

    def store(c):
        return pltpu.async_copy(
            rows[c % 2], out_hbm.at[pl.ds(base + c * _CHUNK, _CHUNK)],
            ssem[c % 2])

    pltpu.sync_copy(idx_hbm.at[pl.ds(base, _CHUNK)], idx_v[0])
    gathers = {0: gather(0)}
    idx_loads = {1: idx_load(1)}
    stores = {}

    for c in range(_NCHUNK):
        b = c % 2
        gathers.pop(c).wait()
        stores[c] = store(c)
        if c + 2 < _NCHUNK:
            idx_loads[c + 2] = idx_load(c + 2)
        if c + 1 < _NCHUNK:
            idx_loads.pop(c + 1).wait()
            if c - 1 in stores:
                stores.pop(c - 1).wait()
            gathers[c + 1] = gather(c + 1)

    for c in sorted(stores):
        stores.pop(c).wait()


def kernel(batch, table):
    idx = batch.reshape(_N).astype(jnp.int32)
    out = _gather_kernel(idx, table)
    return out.reshape(_BATCH, _HIST, _DIM)


# speedup vs baseline: 1.6090x; 1.6090x over previous
import functools

import jax
import jax.numpy as jnp
from jax import lax
from jax.experimental import pallas as pl
from jax.experimental.pallas import tpu as pltpu
from jax.experimental.pallas import tpu_sc as plsc

_BATCH = 16384
_HIST = 50
_DIM = 32
_N = _BATCH * _HIST
_NC = 2
_NS = 16
_NW = _NC * _NS
_PER_W = _N // _NW
_CHUNK = 1600
_NCHUNK = _PER_W // _CHUNK


_CB = _CHUNK // _HIST


@functools.partial(
    pl.kernel,
    out_type=jax.ShapeDtypeStruct((_BATCH, _HIST, _DIM), jnp.float32),
    mesh=plsc.VectorSubcoreMesh(core_axis_name="c", subcore_axis_name="s"),
    scratch_types=[
        pltpu.VMEM((_CHUNK,), jnp.int32),
        pltpu.VMEM((_CHUNK,), jnp.int32),
        pltpu.VMEM((_CHUNK, _DIM), jnp.float32),
        pltpu.VMEM((_CHUNK, _DIM), jnp.float32),
        pltpu.SemaphoreType.DMA,
        pltpu.SemaphoreType.DMA,
        pltpu.SemaphoreType.DMA,
        pltpu.SemaphoreType.DMA,
        pltpu.SemaphoreType.DMA,
        pltpu.SemaphoreType.DMA,
    ],
    compiler_params=pltpu.CompilerParams(use_tc_tiling_on_sc=False),
)
def _gather_kernel(idx_hbm, table_hbm, out_hbm, idx0, idx1, rows0, rows1,
                   i0, i1, g0, g1, s0, s1):
    wid = lax.axis_index("s") * _NC + lax.axis_index("c")
    base = wid * _PER_W
    idx_v = (idx0, idx1)
    rows = (rows0, rows1)
    isem = (i0, i1)
    gsem = (g0, g1)
    ssem = (s0, s1)

    def idx_load(c):
        return pltpu.async_copy(
            idx_hbm.at[pl.ds(base + c * _CHUNK, _CHUNK)],
            idx_v[c % 2], isem[c % 2])

    def gather(c):
        return pltpu.async_copy(
            table_hbm.at[idx_v[c % 2]], rows[c % 2], gsem[c % 2])

    def store(c):
        bat = (base + c * _CHUNK) // _HIST
        return [
            pltpu.async_copy(
                rows[c % 2].at[pl.ds(k * _HIST, _HIST)],
                out_hbm.at[bat + k], ssem[c % 2])
            for k in range(_CB)
        ]

    pltpu.sync_copy(idx_hbm.at[pl.ds(base, _CHUNK)], idx_v[0])
    gathers = {0: gather(0)}
    idx_loads = {1: idx_load(1)}
    stores = {}

    for c in range(_NCHUNK):
        b = c % 2
        gathers.pop(c).wait()
        stores[c] = store(c)
        if c + 2 < _NCHUNK:
            idx_loads[c + 2] = idx_load(c + 2)
        if c + 1 < _NCHUNK:
            idx_loads.pop(c + 1).wait()
            if c - 1 in stores:
                for cp in stores.pop(c - 1):
                    cp.wait()
            gathers[c + 1] = gather(c + 1)

    for c in sorted(stores):
        for cp in stores.pop(c):
            cp.wait()


def kernel(batch, table):
    idx = batch.reshape(_N).astype(jnp.int32)
    return _gather_kernel(idx, table)
